# baseline (speedup 1.0000x reference)
import jax
import jax.numpy as jnp
from jax import lax
from jax.experimental import pallas as pl
from jax.experimental.pallas import tpu as pltpu

_B = 1024
_N = 100000
_L = 50
_LANES = 128
_SUB = 64
_ROWS = 832
_BIG = 0x7FFFFFFF
_KS1 = 42
_KS2 = 42 ^ 0x1BD11BDA


def _threefry_pair(x0, x1):
    ks = (jnp.uint32(0), jnp.uint32(_KS1), jnp.uint32(_KS2))
    rots = ((13, 15, 26, 6), (17, 29, 16, 24))
    x0 = x0 + ks[0]
    x1 = x1 + ks[1]
    for i in range(5):
        for r in rots[i % 2]:
            x0 = x0 + x1
            x1 = (x1 << r) | (x1 >> (32 - r))
            x1 = x1 ^ x0
        x0 = x0 + ks[(i + 1) % 3]
        x1 = x1 + ks[(i + 2) % 3] + jnp.uint32(i + 1)
    return x0, x1


def _accum_loss_term(pos_scr, neg_scr, lanes_scr, acc_ref):
    sub8 = lax.broadcasted_iota(jnp.int32, (8, _LANES), 0)
    lane8 = lax.broadcasted_iota(jnp.int32, (8, _LANES), 1)
    rin = lanes_scr[2]
    pos = jnp.sum(jnp.where((sub8 == rin) & (lane8 == lanes_scr[0]),
                            pos_scr[:, :], jnp.float32(0.0)))
    neg = jnp.sum(jnp.where((sub8 == rin) & (lane8 == lanes_scr[1]),
                            neg_scr[:, :], jnp.float32(0.0)))
    x = jnp.broadcast_to(pos - neg, (1, _LANES))
    ls = jnp.minimum(x, jnp.float32(0.0)) - jnp.log(1.0 + jnp.exp(-jnp.abs(x)))
    acc_ref[:, :] = acc_ref[:, :] + ls


def _fused_body(iseq_ref, tgt_ref, logits_ref, loss_ref,
                mant_ref, j_ref, pos_scr, neg_scr, lanes_scr, acc_ref,
                sem_p, sem_n):
    b = pl.program_id(0)

    @pl.when(b == 0)
    def _():
        sub = lax.broadcasted_iota(jnp.int32, (_ROWS, _LANES), 0)
        lane = lax.broadcasted_iota(jnp.int32, (_ROWS, _LANES), 1)
        j_ref[:, :] = sub * _LANES + lane
        acc_ref[:, :] = jnp.zeros((1, _LANES), jnp.float32)

    @pl.when(b > 0)
    def _():
        pltpu.make_async_copy(
            logits_ref.at[pl.ds(0, 8), pl.ds(0, _LANES)], pos_scr,
            sem_p).wait()
        pltpu.make_async_copy(
            logits_ref.at[pl.ds(0, 8), pl.ds(0, _LANES)], neg_scr,
            sem_n).wait()
        _accum_loss_term(pos_scr, neg_scr, lanes_scr, acc_ref)

    base = (b * _N).astype(jnp.uint32)

    def chunk(i, carry):
        r0 = i * _SUB
        j = j_ref[pl.ds(r0, _SUB), :]
        x1 = j.astype(jnp.uint32) + base
        x0 = jnp.zeros((_SUB, _LANES), jnp.uint32)
        x0, x1 = _threefry_pair(x0, x1)
        mant = ((x0 ^ x1) >> 9).astype(jnp.int32)
        mant_ref[pl.ds(r0, _SUB), :] = jnp.where(j < _N, mant, jnp.int32(-1))
        return carry

    lax.fori_loop(0, _ROWS // _SUB, chunk, 0)

    tgt = tgt_ref[0, 0, 0]
    iseq = iseq_ref[0]

    def reduce_argmax():
        mv = mant_ref[:, :]
        m = jnp.max(mv)
        return jnp.min(jnp.where(mv == m, j_ref[:, :], jnp.int32(_BIG)))

    def is_banned(idx):
        return jnp.any(iseq == idx) | (idx == tgt)

    idx0 = reduce_argmax()

    def cond(c):
        return c[1]

    def body(c):
        idx_p, _ = c
        r = idx_p >> 7
        cc = idx_p & 127
        lane1 = lax.broadcasted_iota(jnp.int32, (1, _LANES), 1)
        row = mant_ref[pl.ds(r, 1), :]
        mant_ref[pl.ds(r, 1), :] = jnp.where(lane1 == cc, jnp.int32(-1), row)
        idx = reduce_argmax()
        return (idx, is_banned(idx))

    idx_f, _ = lax.while_loop(cond, body, (idx0, is_banned(idx0)))

    tstart = pl.multiple_of((tgt >> 7) << 7, _LANES)
    nstart = pl.multiple_of((idx_f >> 7) << 7, _LANES)
    rb = pl.multiple_of((b >> 3) << 3, 8)
    lanes_scr[0] = tgt - tstart
    lanes_scr[1] = idx_f - nstart
    lanes_scr[2] = b & 7
    pltpu.make_async_copy(
        logits_ref.at[pl.ds(rb, 8), pl.ds(tstart, _LANES)], pos_scr,
        sem_p).start()
    pltpu.make_async_copy(
        logits_ref.at[pl.ds(rb, 8), pl.ds(nstart, _LANES)], neg_scr,
        sem_n).start()

    @pl.when(b == _B - 1)
    def _():
        pltpu.make_async_copy(
            logits_ref.at[pl.ds(0, 8), pl.ds(0, _LANES)], pos_scr,
            sem_p).wait()
        pltpu.make_async_copy(
            logits_ref.at[pl.ds(0, 8), pl.ds(0, _LANES)], neg_scr,
            sem_n).wait()
        _accum_loss_term(pos_scr, neg_scr, lanes_scr, acc_ref)
        loss_ref[0, 0] = -jnp.max(acc_ref[:, :]) * jnp.float32(1.0 / _B)


def _make_call():
    return pl.pallas_call(
        _fused_body,
        grid=(_B,),
        in_specs=[
            pl.BlockSpec((1, 1, _L), lambda b: (b, 0, 0)),
            pl.BlockSpec((1, 1, 1), lambda b: (b, 0, 0),
                         memory_space=pltpu.SMEM),
            pl.BlockSpec(memory_space=pl.ANY),
        ],
        out_specs=pl.BlockSpec((1, 1), lambda b: (0, 0),
                               memory_space=pltpu.SMEM),
        out_shape=jax.ShapeDtypeStruct((1, 1), jnp.float32),
        scratch_shapes=[
            pltpu.VMEM((_ROWS, _LANES), jnp.int32),
            pltpu.VMEM((_ROWS, _LANES), jnp.int32),
            pltpu.VMEM((8, _LANES), jnp.float32),
            pltpu.VMEM((8, _LANES), jnp.float32),
            pltpu.SMEM((3,), jnp.int32),
            pltpu.VMEM((1, _LANES), jnp.float32),
            pltpu.SemaphoreType.DMA,
            pltpu.SemaphoreType.DMA,
        ],
    )


def kernel(logits, item_seq, target):
    iseq3 = item_seq.astype(jnp.int32).reshape(_B, 1, _L)
    tgt3 = target.astype(jnp.int32).reshape(_B, 1, 1)
    loss = _make_call()(iseq3, tgt3, logits)
    return loss.reshape(())


# speedup vs baseline: 1.2254x; 1.2254x over previous
import jax
import jax.numpy as jnp
from jax import lax
from jax.experimental import pallas as pl
from jax.experimental.pallas import tpu as pltpu

_B = 1024
_N = 100000
_L = 50
_LANES = 128
_SUB = 64
_ROWS = 832
_BIG = 0x7FFFFFFF
_KS1 = 42
_KS2 = 42 ^ 0x1BD11BDA


def _threefry_pair(x0, x1):
    ks = (jnp.uint32(0), jnp.uint32(_KS1), jnp.uint32(_KS2))
    rots = ((13, 15, 26, 6), (17, 29, 16, 24))
    x0 = x0 + ks[0]
    x1 = x1 + ks[1]
    for i in range(5):
        for r in rots[i % 2]:
            x0 = x0 + x1
            x1 = (x1 << r) | (x1 >> (32 - r))
            x1 = x1 ^ x0
        x0 = x0 + ks[(i + 1) % 3]
        x1 = x1 + ks[(i + 2) % 3] + jnp.uint32(i + 1)
    return x0, x1


def _accum_loss_term(pos_scr, neg_scr, lanes_scr, acc_ref):
    sub8 = lax.broadcasted_iota(jnp.int32, (8, _LANES), 0)
    lane8 = lax.broadcasted_iota(jnp.int32, (8, _LANES), 1)
    rin = lanes_scr[2]
    pos = jnp.sum(jnp.where((sub8 == rin) & (lane8 == lanes_scr[0]),
                            pos_scr[:, :], jnp.float32(0.0)))
    neg = jnp.sum(jnp.where((sub8 == rin) & (lane8 == lanes_scr[1]),
                            neg_scr[:, :], jnp.float32(0.0)))
    x = jnp.broadcast_to(pos - neg, (1, _LANES))
    ls = jnp.minimum(x, jnp.float32(0.0)) - jnp.log(1.0 + jnp.exp(-jnp.abs(x)))
    acc_ref[:, :] = acc_ref[:, :] + ls


def _fused_body(iseq_ref, tgt_ref, logits_ref, loss_ref,
                mant_ref, j_ref, pos_scr, neg_scr, lanes_scr, acc_ref,
                sem_p, sem_n):
    b = pl.program_id(0)

    @pl.when(b == 0)
    def _():
        sub = lax.broadcasted_iota(jnp.int32, (_ROWS, _LANES), 0)
        lane = lax.broadcasted_iota(jnp.int32, (_ROWS, _LANES), 1)
        j_ref[:, :] = sub * _LANES + lane
        acc_ref[:, :] = jnp.zeros((1, _LANES), jnp.float32)

    base = (b * _N).astype(jnp.uint32)

    def chunk(i, carry):
        r0 = i * _SUB
        j = j_ref[pl.ds(r0, _SUB), :]
        x1 = j.astype(jnp.uint32) + base
        x0 = jnp.zeros((_SUB, _LANES), jnp.uint32)
        x0, x1 = _threefry_pair(x0, x1)
        mant = ((x0 ^ x1) >> 9).astype(jnp.int32)
        mant_ref[pl.ds(r0, _SUB), :] = jnp.where(j < _N, mant, jnp.int32(-1))
        return carry

    lax.fori_loop(0, _ROWS // _SUB, chunk, 0)

    tgt = tgt_ref[0, 0, 0]
    iseq = iseq_ref[0]

    def reduce_argmax():
        mv = mant_ref[:, :]
        m = jnp.max(mv)
        return jnp.min(jnp.where(mv == m, j_ref[:, :], jnp.int32(_BIG)))

    def is_banned(idx):
        return jnp.any(iseq == idx) | (idx == tgt)

    idx0 = reduce_argmax()

    def cond(c):
        return c[1]

    def body(c):
        idx_p, _ = c
        r = idx_p >> 7
        cc = idx_p & 127
        lane1 = lax.broadcasted_iota(jnp.int32, (1, _LANES), 1)
        row = mant_ref[pl.ds(r, 1), :]
        mant_ref[pl.ds(r, 1), :] = jnp.where(lane1 == cc, jnp.int32(-1), row)
        idx = reduce_argmax()
        return (idx, is_banned(idx))

    idx_f, _ = lax.while_loop(cond, body, (idx0, is_banned(idx0)))

    @pl.when(b > 0)
    def _():
        pltpu.make_async_copy(
            logits_ref.at[pl.ds(0, 8), pl.ds(0, _LANES)], pos_scr,
            sem_p).wait()
        pltpu.make_async_copy(
            logits_ref.at[pl.ds(0, 8), pl.ds(0, _LANES)], neg_scr,
            sem_n).wait()
        _accum_loss_term(pos_scr, neg_scr, lanes_scr, acc_ref)

    tstart = pl.multiple_of((tgt >> 7) << 7, _LANES)
    nstart = pl.multiple_of((idx_f >> 7) << 7, _LANES)
    rb = pl.multiple_of((b >> 3) << 3, 8)
    lanes_scr[0] = tgt - tstart
    lanes_scr[1] = idx_f - nstart
    lanes_scr[2] = b & 7
    pltpu.make_async_copy(
        logits_ref.at[pl.ds(rb, 8), pl.ds(tstart, _LANES)], pos_scr,
        sem_p).start()
    pltpu.make_async_copy(
        logits_ref.at[pl.ds(rb, 8), pl.ds(nstart, _LANES)], neg_scr,
        sem_n).start()

    @pl.when(b == _B - 1)
    def _():
        pltpu.make_async_copy(
            logits_ref.at[pl.ds(0, 8), pl.ds(0, _LANES)], pos_scr,
            sem_p).wait()
        pltpu.make_async_copy(
            logits_ref.at[pl.ds(0, 8), pl.ds(0, _LANES)], neg_scr,
            sem_n).wait()
        _accum_loss_term(pos_scr, neg_scr, lanes_scr, acc_ref)
        loss_ref[0, 0] = -jnp.max(acc_ref[:, :]) * jnp.float32(1.0 / _B)


def _make_call():
    return pl.pallas_call(
        _fused_body,
        grid=(_B,),
        in_specs=[
            pl.BlockSpec((1, 1, _L), lambda b: (b, 0, 0)),
            pl.BlockSpec((1, 1, 1), lambda b: (b, 0, 0),
                         memory_space=pltpu.SMEM),
            pl.BlockSpec(memory_space=pl.ANY),
        ],
        out_specs=pl.BlockSpec((1, 1), lambda b: (0, 0),
                               memory_space=pltpu.SMEM),
        out_shape=jax.ShapeDtypeStruct((1, 1), jnp.float32),
        scratch_shapes=[
            pltpu.VMEM((_ROWS, _LANES), jnp.int32),
            pltpu.VMEM((_ROWS, _LANES), jnp.int32),
            pltpu.VMEM((8, _LANES), jnp.float32),
            pltpu.VMEM((8, _LANES), jnp.float32),
            pltpu.SMEM((3,), jnp.int32),
            pltpu.VMEM((1, _LANES), jnp.float32),
            pltpu.SemaphoreType.DMA,
            pltpu.SemaphoreType.DMA,
        ],
    )


def kernel(logits, item_seq, target):
    iseq3 = item_seq.astype(jnp.int32).reshape(_B, 1, _L)
    tgt3 = target.astype(jnp.int32).reshape(_B, 1, 1)
    loss = _make_call()(iseq3, tgt3, logits)
    return loss.reshape(())


# speedup vs baseline: 1.2452x; 1.0162x over previous
import jax
import jax.numpy as jnp
from jax import lax
from jax.experimental import pallas as pl
from jax.experimental.pallas import tpu as pltpu

_B = 1024
_N = 100000
_L = 50
_LANES = 128
_SUB = 64
_ROWS = 832
_BIG = 0x7FFFFFFF
_KS1 = 42
_KS2 = 42 ^ 0x1BD11BDA


def _threefry_pair(x0, x1):
    ks = (jnp.uint32(0), jnp.uint32(_KS1), jnp.uint32(_KS2))
    rots = ((13, 15, 26, 6), (17, 29, 16, 24))
    x0 = x0 + ks[0]
    x1 = x1 + ks[1]
    for i in range(5):
        for r in rots[i % 2]:
            x0 = x0 + x1
            x1 = (x1 << r) | (x1 >> (32 - r))
            x1 = x1 ^ x0
        x0 = x0 + ks[(i + 1) % 3]
        x1 = x1 + ks[(i + 2) % 3] + jnp.uint32(i + 1)
    return x0, x1


def _accum_loss_term(pos_scr, neg_scr, lanes_scr, acc_ref):
    sub8 = lax.broadcasted_iota(jnp.int32, (8, _LANES), 0)
    lane8 = lax.broadcasted_iota(jnp.int32, (8, _LANES), 1)
    rin = lanes_scr[2]
    pos = jnp.sum(jnp.where((sub8 == rin) & (lane8 == lanes_scr[0]),
                            pos_scr[:, :], jnp.float32(0.0)))
    neg = jnp.sum(jnp.where((sub8 == rin) & (lane8 == lanes_scr[1]),
                            neg_scr[:, :], jnp.float32(0.0)))
    x = jnp.broadcast_to(pos - neg, (1, _LANES))
    ls = jnp.minimum(x, jnp.float32(0.0)) - jnp.log(1.0 + jnp.exp(-jnp.abs(x)))
    acc_ref[:, :] = acc_ref[:, :] + ls


def _fused_body(iseq_ref, tgt_ref, logits_ref, loss_ref,
                mant_ref, j_ref, pos_scr, neg_scr, lanes_scr, acc_ref,
                sem_p, sem_n):
    b = pl.program_id(0)

    @pl.when(b == 0)
    def _():
        sub = lax.broadcasted_iota(jnp.int32, (_ROWS, _LANES), 0)
        lane = lax.broadcasted_iota(jnp.int32, (_ROWS, _LANES), 1)
        j_ref[:, :] = sub * _LANES + lane
        acc_ref[:, :] = jnp.zeros((1, _LANES), jnp.float32)

    base = (b * _N).astype(jnp.uint32)
    n_chunks = _ROWS // _SUB

    def chunk_mant(i, masked):
        r0 = i * _SUB
        j = j_ref[pl.ds(r0, _SUB), :]
        x1 = j.astype(jnp.uint32) + base
        x0 = jnp.zeros((_SUB, _LANES), jnp.uint32)
        x0, x1 = _threefry_pair(x0, x1)
        mant = ((x0 ^ x1) >> 9).astype(jnp.int32)
        if masked:
            mant = jnp.where(j < _N, mant, jnp.int32(-1))
        mant_ref[pl.ds(r0, _SUB), :] = mant
        return mant

    def chunk(i, carry):
        mx, cx = carry
        mant = chunk_mant(i, False)
        upd = mant > mx
        return (jnp.where(upd, mant, mx), jnp.where(upd, i, cx))

    init = (jnp.full((_SUB, _LANES), -2, jnp.int32),
            jnp.zeros((_SUB, _LANES), jnp.int32))
    mx, cx = lax.fori_loop(0, n_chunks - 1, chunk, init)
    mant = chunk_mant(n_chunks - 1, True)
    upd = mant > mx
    mx = jnp.where(upd, mant, mx)
    cx = jnp.where(upd, jnp.int32(n_chunks - 1), cx)

    tgt = tgt_ref[0, 0, 0]
    iseq = iseq_ref[0]

    def reduce_argmax():
        mv = mant_ref[:, :]
        m = jnp.max(mv)
        return jnp.min(jnp.where(mv == m, j_ref[:, :], jnp.int32(_BIG)))

    def is_banned(idx):
        return jnp.any(iseq == idx) | (idx == tgt)

    m0 = jnp.max(mx)
    jfull = (cx << 13) + j_ref[pl.ds(0, _SUB), :]
    idx0 = jnp.min(jnp.where(mx == m0, jfull, jnp.int32(_BIG)))

    def cond(c):
        return c[1]

    def body(c):
        idx_p, _ = c
        r = idx_p >> 7
        cc = idx_p & 127
        lane1 = lax.broadcasted_iota(jnp.int32, (1, _LANES), 1)
        row = mant_ref[pl.ds(r, 1), :]
        mant_ref[pl.ds(r, 1), :] = jnp.where(lane1 == cc, jnp.int32(-1), row)
        idx = reduce_argmax()
        return (idx, is_banned(idx))

    idx_f, _ = lax.while_loop(cond, body, (idx0, is_banned(idx0)))

    @pl.when(b > 0)
    def _():
        pltpu.make_async_copy(
            logits_ref.at[pl.ds(0, 8), pl.ds(0, _LANES)], pos_scr,
            sem_p).wait()
        pltpu.make_async_copy(
            logits_ref.at[pl.ds(0, 8), pl.ds(0, _LANES)], neg_scr,
            sem_n).wait()
        _accum_loss_term(pos_scr, neg_scr, lanes_scr, acc_ref)

    tstart = pl.multiple_of((tgt >> 7) << 7, _LANES)
    nstart = pl.multiple_of((idx_f >> 7) << 7, _LANES)
    rb = pl.multiple_of((b >> 3) << 3, 8)
    lanes_scr[0] = tgt - tstart
    lanes_scr[1] = idx_f - nstart
    lanes_scr[2] = b & 7
    pltpu.make_async_copy(
        logits_ref.at[pl.ds(rb, 8), pl.ds(tstart, _LANES)], pos_scr,
        sem_p).start()
    pltpu.make_async_copy(
        logits_ref.at[pl.ds(rb, 8), pl.ds(nstart, _LANES)], neg_scr,
        sem_n).start()

    @pl.when(b == _B - 1)
    def _():
        pltpu.make_async_copy(
            logits_ref.at[pl.ds(0, 8), pl.ds(0, _LANES)], pos_scr,
            sem_p).wait()
        pltpu.make_async_copy(
            logits_ref.at[pl.ds(0, 8), pl.ds(0, _LANES)], neg_scr,
            sem_n).wait()
        _accum_loss_term(pos_scr, neg_scr, lanes_scr, acc_ref)
        loss_ref[0, 0] = -jnp.max(acc_ref[:, :]) * jnp.float32(1.0 / _B)


def _make_call():
    return pl.pallas_call(
        _fused_body,
        grid=(_B,),
        in_specs=[
            pl.BlockSpec((1, 1, _L), lambda b: (b, 0, 0)),
            pl.BlockSpec((1, 1, 1), lambda b: (b, 0, 0),
                         memory_space=pltpu.SMEM),
            pl.BlockSpec(memory_space=pl.ANY),
        ],
        out_specs=pl.BlockSpec((1, 1), lambda b: (0, 0),
                               memory_space=pltpu.SMEM),
        out_shape=jax.ShapeDtypeStruct((1, 1), jnp.float32),
        scratch_shapes=[
            pltpu.VMEM((_ROWS, _LANES), jnp.int32),
            pltpu.VMEM((_ROWS, _LANES), jnp.int32),
            pltpu.VMEM((8, _LANES), jnp.float32),
            pltpu.VMEM((8, _LANES), jnp.float32),
            pltpu.SMEM((3,), jnp.int32),
            pltpu.VMEM((1, _LANES), jnp.float32),
            pltpu.SemaphoreType.DMA,
            pltpu.SemaphoreType.DMA,
        ],
    )


def kernel(logits, item_seq, target):
    iseq3 = item_seq.astype(jnp.int32).reshape(_B, 1, _L)
    tgt3 = target.astype(jnp.int32).reshape(_B, 1, 1)
    loss = _make_call()(iseq3, tgt3, logits)
    return loss.reshape(())


# speedup vs baseline: 1.2552x; 1.0081x over previous
import jax
import jax.numpy as jnp
from jax import lax
from jax.experimental import pallas as pl
from jax.experimental.pallas import tpu as pltpu

_B = 1024
_N = 100000
_L = 50
_LANES = 128
_SUB = 64
_ROWS = 832
_NCH = _ROWS // _SUB
_RPG = 8
_GRID = _B // _RPG
_BIG = 0x7FFFFFFF
_KS1 = 42
_KS2 = 42 ^ 0x1BD11BDA


def _threefry_pair(x0, x1):
    ks = (jnp.uint32(0), jnp.uint32(_KS1), jnp.uint32(_KS2))
    rots = ((13, 15, 26, 6), (17, 29, 16, 24))
    x0 = x0 + ks[0]
    x1 = x1 + ks[1]
    for i in range(5):
        for r in rots[i % 2]:
            x0 = x0 + x1
            x1 = (x1 << r) | (x1 >> (32 - r))
            x1 = x1 ^ x0
        x0 = x0 + ks[(i + 1) % 3]
        x1 = x1 + ks[(i + 2) % 3] + jnp.uint32(i + 1)
    return x0, x1


def _accum_loss_term(pos_scr, neg_scr, lanes_scr, acc_ref):
    sub8 = lax.broadcasted_iota(jnp.int32, (8, _LANES), 0)
    lane8 = lax.broadcasted_iota(jnp.int32, (8, _LANES), 1)
    rin = lanes_scr[2]
    pos = jnp.sum(jnp.where((sub8 == rin) & (lane8 == lanes_scr[0]),
                            pos_scr[:, :], jnp.float32(0.0)))
    neg = jnp.sum(jnp.where((sub8 == rin) & (lane8 == lanes_scr[1]),
                            neg_scr[:, :], jnp.float32(0.0)))
    x = jnp.broadcast_to(pos - neg, (1, _LANES))
    ls = jnp.minimum(x, jnp.float32(0.0)) - jnp.log(1.0 + jnp.exp(-jnp.abs(x)))
    acc_ref[:, :] = acc_ref[:, :] + ls


def _fused_body(ban_ref, tgt_ref, logits_ref, loss_ref,
                mant_ref, j_ref, pos_scr, neg_scr, lanes_scr, acc_ref,
                sem_p, sem_n):
    g = pl.program_id(0)

    @pl.when(g == 0)
    def _():
        sub = lax.broadcasted_iota(jnp.int32, (_ROWS, _LANES), 0)
        lane = lax.broadcasted_iota(jnp.int32, (_ROWS, _LANES), 1)
        j_ref[:, :] = sub * _LANES + lane
        acc_ref[:, :] = jnp.zeros((1, _LANES), jnp.float32)

    def row_body(k, carry):
        b = g * _RPG + k
        base = (b * _N).astype(jnp.uint32)

        def chunk_mant(i, masked):
            r0 = i * _SUB
            j = j_ref[pl.ds(r0, _SUB), :]
            x1 = j.astype(jnp.uint32) + base
            x0 = jnp.zeros((_SUB, _LANES), jnp.uint32)
            x0, x1 = _threefry_pair(x0, x1)
            mant = ((x0 ^ x1) >> 9).astype(jnp.int32)
            if masked:
                mant = jnp.where(j < _N, mant, jnp.int32(-1))
            mant_ref[pl.ds(r0, _SUB), :] = mant
            return mant

        def chunk(i, c):
            mx, cx = c
            mant = chunk_mant(i, False)
            upd = mant > mx
            return (jnp.where(upd, mant, mx), jnp.where(upd, i, cx))

        init = (jnp.full((_SUB, _LANES), -2, jnp.int32),
                jnp.zeros((_SUB, _LANES), jnp.int32))
        mx, cx = lax.fori_loop(0, _NCH - 1, chunk, init)
        mant = chunk_mant(_NCH - 1, True)
        upd = mant > mx
        mx = jnp.where(upd, mant, mx)
        cx = jnp.where(upd, jnp.int32(_NCH - 1), cx)

        tgt = tgt_ref[b, 0, 0]
        ban = ban_ref[pl.ds(b, 1), :]

        def reduce_argmax():
            mv = mant_ref[:, :]
            m = jnp.max(mv)
            return jnp.min(jnp.where(mv == m, j_ref[:, :], jnp.int32(_BIG)))

        def is_banned(idx):
            return jnp.any(ban == idx)

        m0 = jnp.max(mx)
        jfull = (cx << 13) + j_ref[pl.ds(0, _SUB), :]
        idx0 = jnp.min(jnp.where(mx == m0, jfull, jnp.int32(_BIG)))

        def cond(c):
            return c[1]

        def body(c):
            idx_p, _ = c
            r = idx_p >> 7
            cc = idx_p & 127
            lane1 = lax.broadcasted_iota(jnp.int32, (1, _LANES), 1)
            row = mant_ref[pl.ds(r, 1), :]
            mant_ref[pl.ds(r, 1), :] = jnp.where(lane1 == cc, jnp.int32(-1),
                                                 row)
            idx = reduce_argmax()
            return (idx, is_banned(idx))

        idx_f, _ = lax.while_loop(cond, body, (idx0, is_banned(idx0)))

        @pl.when(b > 0)
        def _():
            pltpu.make_async_copy(
                logits_ref.at[pl.ds(0, 8), pl.ds(0, _LANES)], pos_scr,
                sem_p).wait()
            pltpu.make_async_copy(
                logits_ref.at[pl.ds(0, 8), pl.ds(0, _LANES)], neg_scr,
                sem_n).wait()
            _accum_loss_term(pos_scr, neg_scr, lanes_scr, acc_ref)

        tstart = pl.multiple_of((tgt >> 7) << 7, _LANES)
        nstart = pl.multiple_of((idx_f >> 7) << 7, _LANES)
        rb = pl.multiple_of((b >> 3) << 3, 8)
        lanes_scr[0] = tgt - tstart
        lanes_scr[1] = idx_f - nstart
        lanes_scr[2] = b & 7
        pltpu.make_async_copy(
            logits_ref.at[pl.ds(rb, 8), pl.ds(tstart, _LANES)], pos_scr,
            sem_p).start()
        pltpu.make_async_copy(
            logits_ref.at[pl.ds(rb, 8), pl.ds(nstart, _LANES)], neg_scr,
            sem_n).start()
        return carry

    lax.fori_loop(0, _RPG, row_body, 0)

    @pl.when(g == _GRID - 1)
    def _():
        pltpu.make_async_copy(
            logits_ref.at[pl.ds(0, 8), pl.ds(0, _LANES)], pos_scr,
            sem_p).wait()
        pltpu.make_async_copy(
            logits_ref.at[pl.ds(0, 8), pl.ds(0, _LANES)], neg_scr,
            sem_n).wait()
        _accum_loss_term(pos_scr, neg_scr, lanes_scr, acc_ref)
        loss_ref[0, 0] = -jnp.max(acc_ref[:, :]) * jnp.float32(1.0 / _B)


def _make_specs():
    in_specs = [
        pl.BlockSpec((_B, 64), lambda g: (0, 0)),
        pl.BlockSpec((_B, 1, 1), lambda g: (0, 0, 0),
                     memory_space=pltpu.SMEM),
        pl.BlockSpec(memory_space=pl.ANY),
    ]
    out_specs = pl.BlockSpec((1, 1), lambda g: (0, 0),
                             memory_space=pltpu.SMEM)
    scratch_shapes = [
        pltpu.VMEM((_ROWS, _LANES), jnp.int32),
        pltpu.VMEM((_ROWS, _LANES), jnp.int32),
        pltpu.VMEM((8, _LANES), jnp.float32),
        pltpu.VMEM((8, _LANES), jnp.float32),
        pltpu.SMEM((3,), jnp.int32),
        pltpu.VMEM((1, _LANES), jnp.float32),
        pltpu.SemaphoreType.DMA,
        pltpu.SemaphoreType.DMA,
    ]
    return in_specs, out_specs, scratch_shapes


def _make_call():
    in_specs, out_specs, scratch_shapes = _make_specs()
    return pl.pallas_call(
        _fused_body,
        grid=(_GRID,),
        in_specs=in_specs,
        out_specs=out_specs,
        out_shape=jax.ShapeDtypeStruct((1, 1), jnp.float32),
        scratch_shapes=scratch_shapes,
    )


def kernel(logits, item_seq, target):
    iseq = item_seq.astype(jnp.int32)
    tgt = target.astype(jnp.int32)
    ban = jnp.concatenate(
        [iseq, tgt[:, None], jnp.full((_B, 13), -1, jnp.int32)], axis=1)
    tgt3 = tgt.reshape(_B, 1, 1)
    loss = _make_call()(ban, tgt3, logits)
    return loss.reshape(())


# speedup vs baseline: 1.4547x; 1.1589x over previous
import jax
import jax.numpy as jnp
from jax import lax
from jax.experimental import pallas as pl
from jax.experimental.pallas import tpu as pltpu

_B = 1024
_N = 100000
_L = 50
_LANES = 128
_SUB = 64
_ROWS = 832
_NCH = _ROWS // _SUB
_RPG = 8
_GRID = _B // _RPG
_BIG = 0x7FFFFFFF
_KS1 = 42
_KS2 = 42 ^ 0x1BD11BDA


def _threefry_pair(x0, x1):
    ks = (jnp.uint32(0), jnp.uint32(_KS1), jnp.uint32(_KS2))
    rots = ((13, 15, 26, 6), (17, 29, 16, 24))
    x0 = x0 + ks[0]
    x1 = x1 + ks[1]
    for i in range(5):
        for r in rots[i % 2]:
            x0 = x0 + x1
            x1 = (x1 << r) | (x1 >> (32 - r))
            x1 = x1 ^ x0
        x0 = x0 + ks[(i + 1) % 3]
        x1 = x1 + ks[(i + 2) % 3] + jnp.uint32(i + 1)
    return x0, x1


def _accum_loss_term(pos_scr, neg_scr, lanes_scr, acc_ref):
    sub8 = lax.broadcasted_iota(jnp.int32, (8, _LANES), 0)
    lane8 = lax.broadcasted_iota(jnp.int32, (8, _LANES), 1)
    rin = lanes_scr[2]
    pos = jnp.sum(jnp.where((sub8 == rin) & (lane8 == lanes_scr[0]),
                            pos_scr[:, :], jnp.float32(0.0)))
    neg = jnp.sum(jnp.where((sub8 == rin) & (lane8 == lanes_scr[1]),
                            neg_scr[:, :], jnp.float32(0.0)))
    x = jnp.broadcast_to(pos - neg, (1, _LANES))
    ls = jnp.minimum(x, jnp.float32(0.0)) - jnp.log(1.0 + jnp.exp(-jnp.abs(x)))
    acc_ref[:, :] = acc_ref[:, :] + ls


def _fused_body(ban_ref, tgt_ref, logits_ref, loss_ref,
                mant_ref, j_ref, pos_scr, neg_scr, lanes_scr, acc_ref,
                sem_p, sem_n):
    g = pl.program_id(0)

    @pl.when(g == 0)
    def _():
        sub = lax.broadcasted_iota(jnp.int32, (_ROWS, _LANES), 0)
        lane = lax.broadcasted_iota(jnp.int32, (_ROWS, _LANES), 1)
        j_ref[:, :] = sub * _LANES + lane
        acc_ref[:, :] = jnp.zeros((1, _LANES), jnp.float32)

    def row_body(k, carry):
        b = g * _RPG + k
        base = (b * _N).astype(jnp.uint32)

        def chunk_mant(i, masked):
            r0 = i * _SUB
            j = j_ref[pl.ds(r0, _SUB), :]
            x1 = j.astype(jnp.uint32) + base
            x0 = jnp.zeros((_SUB, _LANES), jnp.uint32)
            x0, x1 = _threefry_pair(x0, x1)
            mant = ((x0 ^ x1) >> 9).astype(jnp.int32)
            if masked:
                mant = jnp.where(j < _N, mant, jnp.int32(-1))
            mant_ref[pl.ds(r0, _SUB), :] = mant
            return mant

        def chunk(i, c):
            mx, cx = c
            mant = chunk_mant(i, False)
            upd = mant > mx
            return (jnp.where(upd, mant, mx), jnp.where(upd, i, cx))

        init = (jnp.full((_SUB, _LANES), -2, jnp.int32),
                jnp.zeros((_SUB, _LANES), jnp.int32))
        carry_mc = init
        for ci in range(_NCH - 1):
            carry_mc = chunk(ci, carry_mc)
        mx, cx = carry_mc
        mant = chunk_mant(_NCH - 1, True)
        upd = mant > mx
        mx = jnp.where(upd, mant, mx)
        cx = jnp.where(upd, jnp.int32(_NCH - 1), cx)

        tgt = tgt_ref[b, 0, 0]
        ban = ban_ref[pl.ds(b, 1), :]

        def reduce_argmax():
            mv = mant_ref[:, :]
            m = jnp.max(mv)
            return jnp.min(jnp.where(mv == m, j_ref[:, :], jnp.int32(_BIG)))

        def is_banned(idx):
            return jnp.any(ban == idx)

        m0 = jnp.max(mx)
        jfull = (cx << 13) + j_ref[pl.ds(0, _SUB), :]
        idx0 = jnp.min(jnp.where(mx == m0, jfull, jnp.int32(_BIG)))

        def cond(c):
            return c[1]

        def body(c):
            idx_p, _ = c
            r = idx_p >> 7
            cc = idx_p & 127
            lane1 = lax.broadcasted_iota(jnp.int32, (1, _LANES), 1)
            row = mant_ref[pl.ds(r, 1), :]
            mant_ref[pl.ds(r, 1), :] = jnp.where(lane1 == cc, jnp.int32(-1),
                                                 row)
            idx = reduce_argmax()
            return (idx, is_banned(idx))

        idx_f, _ = lax.while_loop(cond, body, (idx0, is_banned(idx0)))

        @pl.when(b > 0)
        def _():
            pltpu.make_async_copy(
                logits_ref.at[pl.ds(0, 8), pl.ds(0, _LANES)], pos_scr,
                sem_p).wait()
            pltpu.make_async_copy(
                logits_ref.at[pl.ds(0, 8), pl.ds(0, _LANES)], neg_scr,
                sem_n).wait()
            _accum_loss_term(pos_scr, neg_scr, lanes_scr, acc_ref)

        tstart = pl.multiple_of((tgt >> 7) << 7, _LANES)
        nstart = pl.multiple_of((idx_f >> 7) << 7, _LANES)
        rb = pl.multiple_of((b >> 3) << 3, 8)
        lanes_scr[0] = tgt - tstart
        lanes_scr[1] = idx_f - nstart
        lanes_scr[2] = b & 7
        pltpu.make_async_copy(
            logits_ref.at[pl.ds(rb, 8), pl.ds(tstart, _LANES)], pos_scr,
            sem_p).start()
        pltpu.make_async_copy(
            logits_ref.at[pl.ds(rb, 8), pl.ds(nstart, _LANES)], neg_scr,
            sem_n).start()
        return carry

    lax.fori_loop(0, _RPG, row_body, 0)

    @pl.when(g == _GRID - 1)
    def _():
        pltpu.make_async_copy(
            logits_ref.at[pl.ds(0, 8), pl.ds(0, _LANES)], pos_scr,
            sem_p).wait()
        pltpu.make_async_copy(
            logits_ref.at[pl.ds(0, 8), pl.ds(0, _LANES)], neg_scr,
            sem_n).wait()
        _accum_loss_term(pos_scr, neg_scr, lanes_scr, acc_ref)
        loss_ref[0, 0] = -jnp.max(acc_ref[:, :]) * jnp.float32(1.0 / _B)


def _make_specs():
    in_specs = [
        pl.BlockSpec((_B, 64), lambda g: (0, 0)),
        pl.BlockSpec((_B, 1, 1), lambda g: (0, 0, 0),
                     memory_space=pltpu.SMEM),
        pl.BlockSpec(memory_space=pl.ANY),
    ]
    out_specs = pl.BlockSpec((1, 1), lambda g: (0, 0),
                             memory_space=pltpu.SMEM)
    scratch_shapes = [
        pltpu.VMEM((_ROWS, _LANES), jnp.int32),
        pltpu.VMEM((_ROWS, _LANES), jnp.int32),
        pltpu.VMEM((8, _LANES), jnp.float32),
        pltpu.VMEM((8, _LANES), jnp.float32),
        pltpu.SMEM((3,), jnp.int32),
        pltpu.VMEM((1, _LANES), jnp.float32),
        pltpu.SemaphoreType.DMA,
        pltpu.SemaphoreType.DMA,
    ]
    return in_specs, out_specs, scratch_shapes


def _make_call():
    in_specs, out_specs, scratch_shapes = _make_specs()
    return pl.pallas_call(
        _fused_body,
        grid=(_GRID,),
        in_specs=in_specs,
        out_specs=out_specs,
        out_shape=jax.ShapeDtypeStruct((1, 1), jnp.float32),
        scratch_shapes=scratch_shapes,
    )


def kernel(logits, item_seq, target):
    iseq = item_seq.astype(jnp.int32)
    tgt = target.astype(jnp.int32)
    ban = jnp.concatenate(
        [iseq, tgt[:, None], jnp.full((_B, 13), -1, jnp.int32)], axis=1)
    tgt3 = tgt.reshape(_B, 1, 1)
    loss = _make_call()(ban, tgt3, logits)
    return loss.reshape(())
